# baseline (speedup 1.0000x reference)
import functools

import jax
import jax.numpy as jnp
from jax import lax
from jax.experimental import pallas as pl
from jax.experimental.pallas import tpu as pltpu
from jax.experimental.pallas import tpu_sc as plsc

N = 100000
E = 6400000
D = 128
H = 64
G = 64
FIX_R = 0.7


TN = 20000
NB = N // TN


def _tc_body(x_ref, b3_ref, w1_ref, b1_ref, w2_ref, b2_ref, wc_ref, bc_ref,
             lg_ref, clf_ref, seg_acc, cnt_acc):
    i = pl.program_id(0)

    @pl.when(i == 0)
    def _init():
        seg_acc[...] = jnp.zeros_like(seg_acc)
        cnt_acc[...] = jnp.zeros_like(cnt_acc)

    x = x_ref[...]
    h = jnp.maximum(
        jnp.dot(x, w1_ref[...], preferred_element_type=jnp.float32)
        + b1_ref[...], 0.0)
    lg_ref[...] = jnp.dot(h, w2_ref[...],
                          preferred_element_type=jnp.float32) + b2_ref[...]

    bt = b3_ref[0]
    seg_ids = lax.broadcasted_iota(jnp.int32, (G, TN), 0)
    onehot = (seg_ids == bt).astype(jnp.float32)
    seg_acc[...] = seg_acc[...] + jnp.dot(
        onehot, x, preferred_element_type=jnp.float32)
    cnt_acc[...] = cnt_acc[...] + jnp.sum(onehot, axis=1, keepdims=True)

    @pl.when(i == NB - 1)
    def _fin():
        pooled = seg_acc[...] / jnp.maximum(cnt_acc[...], 1.0)
        clf_ref[...] = jnp.dot(pooled, wc_ref[...],
                               preferred_element_type=jnp.float32) + bc_ref[...]


_tc_call = pl.pallas_call(
    _tc_body,
    grid=(NB,),
    in_specs=[
        pl.BlockSpec((TN, D), lambda i: (i, 0)),
        pl.BlockSpec((1, 1, TN), lambda i: (i, 0, 0)),
        pl.BlockSpec((D, H), lambda i: (0, 0)),
        pl.BlockSpec((1, H), lambda i: (0, 0)),
        pl.BlockSpec((H, 1), lambda i: (0, 0)),
        pl.BlockSpec((1, 1), lambda i: (0, 0)),
        pl.BlockSpec((D, 1), lambda i: (0, 0)),
        pl.BlockSpec((1, 1), lambda i: (0, 0)),
    ],
    out_specs=[
        pl.BlockSpec((TN, 1), lambda i: (i, 0)),
        pl.BlockSpec((G, 1), lambda i: (0, 0)),
    ],
    out_shape=[
        jax.ShapeDtypeStruct((100096, 1), jnp.float32),
        jax.ShapeDtypeStruct((G, 1), jnp.float32),
    ],
    scratch_shapes=[
        pltpu.VMEM((G, D), jnp.float32),
        pltpu.VMEM((G, 1), jnp.float32),
    ],
)


NP = 100096
WR = NP // 128


def _att_body(lgw_ref, clf_ref, y_ref, att_ref, loss_ref):
    lg = lgw_ref[...]
    att = 1.0 / (1.0 + jnp.exp(-lg))
    att_ref[...] = att
    r = FIX_R
    il = att * jnp.log(att / r + 1e-06) \
        + (1.0 - att) * jnp.log((1.0 - att) / (1.0 - r + 1e-06) + 1e-06)
    flat = (lax.broadcasted_iota(jnp.int32, (WR, 128), 0) * 128
            + lax.broadcasted_iota(jnp.int32, (WR, 128), 1))
    il = jnp.where(flat < N, il, 0.0)
    info = jnp.sum(il) / jnp.float32(N)
    clf = clf_ref[...]
    yf = y_ref[...].astype(jnp.float32)
    pred = jnp.mean(jnp.maximum(clf, 0.0)
                    + jnp.log(1.0 + jnp.exp(-jnp.abs(clf))) - clf * yf)
    loss_ref[...] = (pred + info).reshape(1, 1)


_att_call = pl.pallas_call(
    _att_body,
    out_shape=[
        jax.ShapeDtypeStruct((WR, 128), jnp.float32),
        jax.ShapeDtypeStruct((1, 1), jnp.float32),
    ],
)


NC = 2
NS = 16
NW = NC * NS
C = 5120
M = E // C
TPAIR = (M + 2 * NW - 1) // (2 * NW)


def _gather_chunk(table, idx2, prod):
    @plsc.parallel_loop(0, C, 16, unroll=16)
    def _inner(j):
        vs = idx2[0, pl.ds(j, 16)]
        vd = idx2[1, pl.ds(j, 16)]
        a = plsc.load_gather(table, [vs])
        b = plsc.load_gather(table, [vd])
        prod[pl.ds(j, 16)] = a * b


def _sc_body(att_hbm, ei_hbm, out_hbm, table, ixa, ixb, pra, prb,
             s_tab, s_ia, s_ib, s_oa, s_ob):
    wid = lax.axis_index("s") * NC + lax.axis_index("c")
    tab_cp = pltpu.async_copy(att_hbm, table, s_tab)
    pltpu.async_copy(ei_hbm.at[:, pl.ds(wid * C, C)], ixa, s_ia)
    tab_cp.wait()

    def pair_body(t, carry):
        ga = wid + (2 * t) * NW
        gb = wid + (2 * t + 1) * NW
        gn = wid + (2 * t + 2) * NW

        @pl.when(gb < M)
        def _issue_b():
            pltpu.async_copy(ei_hbm.at[:, pl.ds(gb * C, C)], ixb, s_ib)

        @pl.when(ga < M)
        def _do_a():
            pltpu.make_async_copy(
                ei_hbm.at[:, pl.ds(ga * C, C)], ixa, s_ia).wait()

            @pl.when((t > 0) & (ga < M))
            def _drain_oa():
                pltpu.make_async_copy(
                    pra, out_hbm.at[pl.ds(ga * C, C)], s_oa).wait()

            _gather_chunk(table, ixa, pra)
            pltpu.async_copy(pra, out_hbm.at[pl.ds(ga * C, C)], s_oa)

        @pl.when(gn < M)
        def _issue_next_a():
            pltpu.async_copy(ei_hbm.at[:, pl.ds(gn * C, C)], ixa, s_ia)

        @pl.when(gb < M)
        def _do_b():
            pltpu.make_async_copy(
                ei_hbm.at[:, pl.ds(gb * C, C)], ixb, s_ib).wait()

            @pl.when((t > 0) & (gb < M))
            def _drain_ob():
                pltpu.make_async_copy(
                    prb, out_hbm.at[pl.ds(gb * C, C)], s_ob).wait()

            _gather_chunk(table, ixb, prb)
            pltpu.async_copy(prb, out_hbm.at[pl.ds(gb * C, C)], s_ob)

        return carry

    lax.fori_loop(0, TPAIR, pair_body, 0)

    ga_last = wid + (2 * TPAIR - 2) * NW
    gb_last = wid + (2 * TPAIR - 1) * NW

    @pl.when(ga_last < M)
    def _final_oa():
        pltpu.make_async_copy(
            pra, out_hbm.at[pl.ds(ga_last * C, C)], s_oa).wait()

    @pl.when(gb_last < M)
    def _final_ob():
        pltpu.make_async_copy(
            prb, out_hbm.at[pl.ds(gb_last * C, C)], s_ob).wait()


@functools.cache
def _make_sc_call():
    mesh = plsc.VectorSubcoreMesh(core_axis_name="c", subcore_axis_name="s")
    return pl.kernel(
        _sc_body,
        mesh=mesh,
        compiler_params=pltpu.CompilerParams(needs_layout_passes=False),
        out_type=jax.ShapeDtypeStruct((E,), jnp.float32),
        scratch_types=[
            pltpu.VMEM((NP,), jnp.float32),
            pltpu.VMEM((2, C), jnp.int32),
            pltpu.VMEM((2, C), jnp.int32),
            pltpu.VMEM((C,), jnp.float32),
            pltpu.VMEM((C,), jnp.float32),
            pltpu.SemaphoreType.DMA,
            pltpu.SemaphoreType.DMA,
            pltpu.SemaphoreType.DMA,
            pltpu.SemaphoreType.DMA,
            pltpu.SemaphoreType.DMA,
        ],
    )


def kernel(x, edge_index, batch, y, W1, b1, W2, b2, Wc, bc):
    batch3 = batch.reshape(NB, 1, TN)
    logits, clf_logits = _tc_call(
        x, batch3, W1, b1.reshape(1, H), W2, b2.reshape(1, 1),
        Wc, bc.reshape(1, 1))
    att_w, loss = _att_call(logits.reshape(WR, 128), clf_logits, y)
    edge_att = _make_sc_call()(att_w.reshape(NP), edge_index)
    return edge_att.reshape(E, 1), loss.reshape(()), clf_logits


# speedup vs baseline: 1.0117x; 1.0117x over previous
import functools

import jax
import jax.numpy as jnp
from jax import lax
from jax.experimental import pallas as pl
from jax.experimental.pallas import tpu as pltpu
from jax.experimental.pallas import tpu_sc as plsc

N = 100000
E = 6400000
D = 128
H = 64
G = 64
FIX_R = 0.7


TN = 10000
NB = N // TN


def _tc_body(x_ref, b3_ref, w1_ref, b1_ref, w2_ref, b2_ref, wc_ref, bc_ref,
             lg_ref, clf_ref, seg_acc, cnt_acc):
    i = pl.program_id(0)

    @pl.when(i == 0)
    def _init():
        seg_acc[...] = jnp.zeros_like(seg_acc)
        cnt_acc[...] = jnp.zeros_like(cnt_acc)

    x = x_ref[...]
    h = jnp.maximum(
        jnp.dot(x, w1_ref[...], preferred_element_type=jnp.float32)
        + b1_ref[...], 0.0)
    lg_ref[...] = jnp.dot(h, w2_ref[...],
                          preferred_element_type=jnp.float32) + b2_ref[...]

    bt = b3_ref[0]
    seg_ids = lax.broadcasted_iota(jnp.int32, (G, TN), 0)
    onehot = (seg_ids == bt).astype(jnp.float32)
    seg_acc[...] = seg_acc[...] + jnp.dot(
        onehot, x, preferred_element_type=jnp.float32)
    cnt_acc[...] = cnt_acc[...] + jnp.sum(onehot, axis=1, keepdims=True)

    @pl.when(i == NB - 1)
    def _fin():
        pooled = seg_acc[...] / jnp.maximum(cnt_acc[...], 1.0)
        clf_ref[...] = jnp.dot(pooled, wc_ref[...],
                               preferred_element_type=jnp.float32) + bc_ref[...]


_tc_call = pl.pallas_call(
    _tc_body,
    grid=(NB,),
    in_specs=[
        pl.BlockSpec((TN, D), lambda i: (i, 0)),
        pl.BlockSpec((1, 1, TN), lambda i: (i, 0, 0)),
        pl.BlockSpec((D, H), lambda i: (0, 0)),
        pl.BlockSpec((1, H), lambda i: (0, 0)),
        pl.BlockSpec((H, 1), lambda i: (0, 0)),
        pl.BlockSpec((1, 1), lambda i: (0, 0)),
        pl.BlockSpec((D, 1), lambda i: (0, 0)),
        pl.BlockSpec((1, 1), lambda i: (0, 0)),
    ],
    out_specs=[
        pl.BlockSpec((TN, 1), lambda i: (i, 0)),
        pl.BlockSpec((G, 1), lambda i: (0, 0)),
    ],
    out_shape=[
        jax.ShapeDtypeStruct((100096, 1), jnp.float32),
        jax.ShapeDtypeStruct((G, 1), jnp.float32),
    ],
    scratch_shapes=[
        pltpu.VMEM((G, D), jnp.float32),
        pltpu.VMEM((G, 1), jnp.float32),
    ],
)


NP = 100096
WR = NP // 128


def _att_body(lgw_ref, clf_ref, y_ref, att_ref, loss_ref):
    lg = lgw_ref[...]
    att = 1.0 / (1.0 + jnp.exp(-lg))
    att_ref[...] = att
    r = FIX_R
    il = att * jnp.log(att / r + 1e-06) \
        + (1.0 - att) * jnp.log((1.0 - att) / (1.0 - r + 1e-06) + 1e-06)
    flat = (lax.broadcasted_iota(jnp.int32, (WR, 128), 0) * 128
            + lax.broadcasted_iota(jnp.int32, (WR, 128), 1))
    il = jnp.where(flat < N, il, 0.0)
    info = jnp.sum(il) / jnp.float32(N)
    clf = clf_ref[...]
    yf = y_ref[...].astype(jnp.float32)
    pred = jnp.mean(jnp.maximum(clf, 0.0)
                    + jnp.log(1.0 + jnp.exp(-jnp.abs(clf))) - clf * yf)
    loss_ref[...] = (pred + info).reshape(1, 1)


_att_call = pl.pallas_call(
    _att_body,
    out_shape=[
        jax.ShapeDtypeStruct((WR, 128), jnp.float32),
        jax.ShapeDtypeStruct((1, 1), jnp.float32),
    ],
)


NC = 2
NS = 16
NW = NC * NS
C = 5120
M = E // C
TPAIR = (M + 2 * NW - 1) // (2 * NW)


def _gather_chunk(table, idx2, prod):
    @plsc.parallel_loop(0, C, 16, unroll=16)
    def _inner(j):
        vs = idx2[0, pl.ds(j, 16)]
        vd = idx2[1, pl.ds(j, 16)]
        a = plsc.load_gather(table, [vs])
        b = plsc.load_gather(table, [vd])
        prod[pl.ds(j, 16)] = a * b


def _sc_body(att_hbm, ei_hbm, out_hbm, table, ixa, ixb, pra, prb,
             s_tab, s_ia, s_ib, s_oa, s_ob):
    wid = lax.axis_index("s") * NC + lax.axis_index("c")
    tab_cp = pltpu.async_copy(att_hbm, table, s_tab)
    pltpu.async_copy(ei_hbm.at[:, pl.ds(wid * C, C)], ixa, s_ia)
    tab_cp.wait()

    def pair_body(t, carry):
        ga = wid + (2 * t) * NW
        gb = wid + (2 * t + 1) * NW
        gn = wid + (2 * t + 2) * NW

        @pl.when(gb < M)
        def _issue_b():
            pltpu.async_copy(ei_hbm.at[:, pl.ds(gb * C, C)], ixb, s_ib)

        @pl.when(ga < M)
        def _do_a():
            pltpu.make_async_copy(
                ei_hbm.at[:, pl.ds(ga * C, C)], ixa, s_ia).wait()

            @pl.when((t > 0) & (ga < M))
            def _drain_oa():
                pltpu.make_async_copy(
                    pra, out_hbm.at[pl.ds(ga * C, C)], s_oa).wait()

            _gather_chunk(table, ixa, pra)
            pltpu.async_copy(pra, out_hbm.at[pl.ds(ga * C, C)], s_oa)

        @pl.when(gn < M)
        def _issue_next_a():
            pltpu.async_copy(ei_hbm.at[:, pl.ds(gn * C, C)], ixa, s_ia)

        @pl.when(gb < M)
        def _do_b():
            pltpu.make_async_copy(
                ei_hbm.at[:, pl.ds(gb * C, C)], ixb, s_ib).wait()

            @pl.when((t > 0) & (gb < M))
            def _drain_ob():
                pltpu.make_async_copy(
                    prb, out_hbm.at[pl.ds(gb * C, C)], s_ob).wait()

            _gather_chunk(table, ixb, prb)
            pltpu.async_copy(prb, out_hbm.at[pl.ds(gb * C, C)], s_ob)

        return carry

    lax.fori_loop(0, TPAIR, pair_body, 0)

    ga_last = wid + (2 * TPAIR - 2) * NW
    gb_last = wid + (2 * TPAIR - 1) * NW

    @pl.when(ga_last < M)
    def _final_oa():
        pltpu.make_async_copy(
            pra, out_hbm.at[pl.ds(ga_last * C, C)], s_oa).wait()

    @pl.when(gb_last < M)
    def _final_ob():
        pltpu.make_async_copy(
            prb, out_hbm.at[pl.ds(gb_last * C, C)], s_ob).wait()


@functools.cache
def _make_sc_call():
    mesh = plsc.VectorSubcoreMesh(core_axis_name="c", subcore_axis_name="s")
    return pl.kernel(
        _sc_body,
        mesh=mesh,
        compiler_params=pltpu.CompilerParams(needs_layout_passes=False),
        out_type=jax.ShapeDtypeStruct((E,), jnp.float32),
        scratch_types=[
            pltpu.VMEM((NP,), jnp.float32),
            pltpu.VMEM((2, C), jnp.int32),
            pltpu.VMEM((2, C), jnp.int32),
            pltpu.VMEM((C,), jnp.float32),
            pltpu.VMEM((C,), jnp.float32),
            pltpu.SemaphoreType.DMA,
            pltpu.SemaphoreType.DMA,
            pltpu.SemaphoreType.DMA,
            pltpu.SemaphoreType.DMA,
            pltpu.SemaphoreType.DMA,
        ],
    )


def kernel(x, edge_index, batch, y, W1, b1, W2, b2, Wc, bc):
    batch3 = batch.reshape(NB, 1, TN)
    logits, clf_logits = _tc_call(
        x, batch3, W1, b1.reshape(1, H), W2, b2.reshape(1, 1),
        Wc, bc.reshape(1, 1))
    att_w, loss = _att_call(logits.reshape(WR, 128), clf_logits, y)
    edge_att = _make_sc_call()(att_w.reshape(NP), edge_index)
    return edge_att.reshape(E, 1), loss.reshape(()), clf_logits


# speedup vs baseline: 1.0238x; 1.0119x over previous
import functools

import jax
import jax.numpy as jnp
from jax import lax
from jax.experimental import pallas as pl
from jax.experimental.pallas import tpu as pltpu
from jax.experimental.pallas import tpu_sc as plsc

N = 100000
E = 6400000
D = 128
H = 64
G = 64
FIX_R = 0.7


TN = 10000
NB = N // TN


def _tc_body(x_ref, b3_ref, w1_ref, b1_ref, w2_ref, b2_ref, wc_ref, bc_ref,
             lg_ref, clf_ref, seg_acc, cnt_acc):
    i = pl.program_id(0)

    @pl.when(i == 0)
    def _init():
        seg_acc[...] = jnp.zeros_like(seg_acc)
        cnt_acc[...] = jnp.zeros_like(cnt_acc)

    x = x_ref[...]
    h = jnp.maximum(
        jnp.dot(x, w1_ref[...], preferred_element_type=jnp.float32)
        + b1_ref[...], 0.0)
    lg_ref[...] = jnp.dot(h, w2_ref[...],
                          preferred_element_type=jnp.float32) + b2_ref[...]

    bt = b3_ref[0]
    seg_ids = lax.broadcasted_iota(jnp.int32, (G, TN), 0)
    onehot = (seg_ids == bt).astype(jnp.float32)
    seg_acc[...] = seg_acc[...] + jnp.dot(
        onehot, x, preferred_element_type=jnp.float32)
    cnt_acc[...] = cnt_acc[...] + jnp.sum(onehot, axis=1, keepdims=True)

    @pl.when(i == NB - 1)
    def _fin():
        pooled = seg_acc[...] / jnp.maximum(cnt_acc[...], 1.0)
        clf_ref[...] = jnp.dot(pooled, wc_ref[...],
                               preferred_element_type=jnp.float32) + bc_ref[...]


_tc_call = pl.pallas_call(
    _tc_body,
    grid=(NB,),
    in_specs=[
        pl.BlockSpec((TN, D), lambda i: (i, 0)),
        pl.BlockSpec((1, 1, TN), lambda i: (i, 0, 0)),
        pl.BlockSpec((D, H), lambda i: (0, 0)),
        pl.BlockSpec((1, H), lambda i: (0, 0)),
        pl.BlockSpec((H, 1), lambda i: (0, 0)),
        pl.BlockSpec((1, 1), lambda i: (0, 0)),
        pl.BlockSpec((D, 1), lambda i: (0, 0)),
        pl.BlockSpec((1, 1), lambda i: (0, 0)),
    ],
    out_specs=[
        pl.BlockSpec((TN, 1), lambda i: (i, 0)),
        pl.BlockSpec((G, 1), lambda i: (0, 0)),
    ],
    out_shape=[
        jax.ShapeDtypeStruct((100096, 1), jnp.float32),
        jax.ShapeDtypeStruct((G, 1), jnp.float32),
    ],
    scratch_shapes=[
        pltpu.VMEM((G, D), jnp.float32),
        pltpu.VMEM((G, 1), jnp.float32),
    ],
)


NP = 100096
WR = NP // 128


def _att_body(lgw_ref, att_ref):
    lg = lgw_ref[...]
    att_ref[...] = 1.0 / (1.0 + jnp.exp(-lg))


_att_call = pl.pallas_call(
    _att_body,
    out_shape=jax.ShapeDtypeStruct((WR, 128), jnp.float32),
)


def _loss_body(att_ref, clf_ref, y_ref, loss_ref):
    att = att_ref[...]
    r = FIX_R
    il = att * jnp.log(att / r + 1e-06) \
        + (1.0 - att) * jnp.log((1.0 - att) / (1.0 - r + 1e-06) + 1e-06)
    flat = (lax.broadcasted_iota(jnp.int32, (WR, 128), 0) * 128
            + lax.broadcasted_iota(jnp.int32, (WR, 128), 1))
    il = jnp.where(flat < N, il, 0.0)
    info = jnp.sum(il) / jnp.float32(N)
    clf = clf_ref[...]
    yf = y_ref[...].astype(jnp.float32)
    pred = jnp.mean(jnp.maximum(clf, 0.0)
                    + jnp.log(1.0 + jnp.exp(-jnp.abs(clf))) - clf * yf)
    loss_ref[...] = (pred + info).reshape(1, 1)


_loss_call = pl.pallas_call(
    _loss_body,
    out_shape=jax.ShapeDtypeStruct((1, 1), jnp.float32),
)


NC = 2
NS = 16
NW = NC * NS
C = 5120
M = E // C
TPAIR = (M + 2 * NW - 1) // (2 * NW)


def _gather_chunk(table, idx2, prod):
    @plsc.parallel_loop(0, C, 16, unroll=16)
    def _inner(j):
        vs = idx2[0, pl.ds(j, 16)]
        vd = idx2[1, pl.ds(j, 16)]
        a = plsc.load_gather(table, [vs])
        b = plsc.load_gather(table, [vd])
        prod[pl.ds(j, 16)] = a * b


def _sc_body(att_hbm, ei_hbm, out_hbm, table, ixa, ixb, pra, prb,
             s_tab, s_ia, s_ib, s_oa, s_ob):
    wid = lax.axis_index("s") * NC + lax.axis_index("c")
    tab_cp = pltpu.async_copy(att_hbm, table, s_tab)
    pltpu.async_copy(ei_hbm.at[:, pl.ds(wid * C, C)], ixa, s_ia)
    tab_cp.wait()

    def pair_body(t, carry):
        ga = wid + (2 * t) * NW
        gb = wid + (2 * t + 1) * NW
        gn = wid + (2 * t + 2) * NW

        @pl.when(gb < M)
        def _issue_b():
            pltpu.async_copy(ei_hbm.at[:, pl.ds(gb * C, C)], ixb, s_ib)

        @pl.when(ga < M)
        def _do_a():
            pltpu.make_async_copy(
                ei_hbm.at[:, pl.ds(ga * C, C)], ixa, s_ia).wait()

            @pl.when((t > 0) & (ga < M))
            def _drain_oa():
                pltpu.make_async_copy(
                    pra, out_hbm.at[pl.ds(ga * C, C)], s_oa).wait()

            _gather_chunk(table, ixa, pra)
            pltpu.async_copy(pra, out_hbm.at[pl.ds(ga * C, C)], s_oa)

        @pl.when(gn < M)
        def _issue_next_a():
            pltpu.async_copy(ei_hbm.at[:, pl.ds(gn * C, C)], ixa, s_ia)

        @pl.when(gb < M)
        def _do_b():
            pltpu.make_async_copy(
                ei_hbm.at[:, pl.ds(gb * C, C)], ixb, s_ib).wait()

            @pl.when((t > 0) & (gb < M))
            def _drain_ob():
                pltpu.make_async_copy(
                    prb, out_hbm.at[pl.ds(gb * C, C)], s_ob).wait()

            _gather_chunk(table, ixb, prb)
            pltpu.async_copy(prb, out_hbm.at[pl.ds(gb * C, C)], s_ob)

        return carry

    lax.fori_loop(0, TPAIR, pair_body, 0)

    ga_last = wid + (2 * TPAIR - 2) * NW
    gb_last = wid + (2 * TPAIR - 1) * NW

    @pl.when(ga_last < M)
    def _final_oa():
        pltpu.make_async_copy(
            pra, out_hbm.at[pl.ds(ga_last * C, C)], s_oa).wait()

    @pl.when(gb_last < M)
    def _final_ob():
        pltpu.make_async_copy(
            prb, out_hbm.at[pl.ds(gb_last * C, C)], s_ob).wait()


@functools.cache
def _make_sc_call():
    mesh = plsc.VectorSubcoreMesh(core_axis_name="c", subcore_axis_name="s")
    return pl.kernel(
        _sc_body,
        mesh=mesh,
        compiler_params=pltpu.CompilerParams(needs_layout_passes=False),
        out_type=jax.ShapeDtypeStruct((E,), jnp.float32),
        scratch_types=[
            pltpu.VMEM((NP,), jnp.float32),
            pltpu.VMEM((2, C), jnp.int32),
            pltpu.VMEM((2, C), jnp.int32),
            pltpu.VMEM((C,), jnp.float32),
            pltpu.VMEM((C,), jnp.float32),
            pltpu.SemaphoreType.DMA,
            pltpu.SemaphoreType.DMA,
            pltpu.SemaphoreType.DMA,
            pltpu.SemaphoreType.DMA,
            pltpu.SemaphoreType.DMA,
        ],
    )


def kernel(x, edge_index, batch, y, W1, b1, W2, b2, Wc, bc):
    batch3 = batch.reshape(NB, 1, TN)
    logits, clf_logits = _tc_call(
        x, batch3, W1, b1.reshape(1, H), W2, b2.reshape(1, 1),
        Wc, bc.reshape(1, 1))
    att_w = _att_call(logits.reshape(WR, 128))
    edge_att = _make_sc_call()(att_w.reshape(NP), edge_index)
    loss = _loss_call(att_w, clf_logits, y)
    return edge_att.reshape(E, 1), loss.reshape(()), clf_logits
